# baseline (speedup 1.0000x reference)
import jax
import jax.numpy as jnp
from jax import lax
from jax.experimental import pallas as pl
from jax.experimental.pallas import tpu as pltpu
from jax.experimental.pallas import tpu_sc as plsc

N_IN = 500000
OUT = 1000000
C = 64
CP = 128
NW = 32
RPW = 31248
ZR = 124
NZ = RPW // ZR
TAIL = OUT - NW * RPW
IB = 100
GW = 8
WR = IB * GW
NG = N_IN // WR
SB = 48


def _sc_body(x_hbm, idx2_hbm, starts_hbm, out_hbm,
             zeros_v, idxw_v, xw_v, starts_s, sem_z):
    c = lax.axis_index("c")
    s = lax.axis_index("s")
    wid = s * 2 + c
    base = wid * RPW

    pltpu.sync_copy(starts_hbm, starts_s)

    zvec = jnp.zeros((16,), jnp.float32)

    @pl.loop(0, ZR)
    def _(r):
        for j in range(C // 16):
            zeros_v[r, pl.ds(j * 16, 16)] = zvec

    zcopies = [
        pltpu.async_copy(
            zeros_v,
            out_hbm.at[pl.ds(base + k * ZR, ZR), pl.ds(0, C)],
            sem_z,
        )
        for k in range(NZ)
    ]
    for cp in zcopies:
        cp.wait()

    @pl.when(wid == NW - 1)
    def _():
        pltpu.async_copy(
            zeros_v.at[pl.ds(0, TAIL)],
            out_hbm.at[pl.ds(NW * RPW, TAIL), pl.ds(0, C)],
            sem_z,
        ).wait()

    sv = starts_s[pl.ds(wid, 16)]
    lo = sv[0]
    hi = sv[1]
    g0 = lo // WR
    g1 = (hi + WR - 1) // WR

    def win(g, carry):
        pltpu.sync_copy(idx2_hbm.at[pl.ds(g * GW, GW)], idxw_v)
        pltpu.sync_copy(x_hbm.at[pl.ds(g * WR, WR), pl.ds(0, C)],
                        xw_v.at[pl.ds(0, WR), pl.ds(0, C)])
        for j in range(GW):
            pltpu.sync_copy(xw_v.at[pl.ds(j * IB, IB)],
                            out_hbm.at[idxw_v.at[j]])
        return carry

    lax.fori_loop(g0, g1, win, 0)


def kernel(x, idx, out_size):
    del out_size
    idx = idx.astype(jnp.int32)
    bounds = jnp.concatenate([
        jnp.arange(0, NW * RPW, RPW, dtype=jnp.int32),
        jnp.array([OUT], dtype=jnp.int32),
    ])
    starts = jnp.searchsorted(idx, bounds).astype(jnp.int32)
    starts = jnp.zeros((SB,), jnp.int32).at[: NW + 1].set(starts)
    idx2 = idx.reshape(NG * GW, IB)

    mesh = plsc.VectorSubcoreMesh(core_axis_name="c", subcore_axis_name="s")
    xp = jnp.pad(x, ((0, 0), (0, CP - C)))
    run = pl.kernel(
        _sc_body,
        out_type=jax.ShapeDtypeStruct((OUT, CP), jnp.float32),
        mesh=mesh,
        compiler_params=pltpu.CompilerParams(use_tc_tiling_on_sc=False),
        scratch_types=[
            pltpu.VMEM((ZR, C), jnp.float32),
            pltpu.VMEM((GW, IB), jnp.int32),
            pltpu.VMEM((WR, CP), jnp.float32),
            pltpu.VMEM((SB,), jnp.int32),
            pltpu.SemaphoreType.DMA,
        ],
    )
    outp = run(xp, idx2, starts)
    return outp[:, :C]


# speedup vs baseline: 1.0294x; 1.0294x over previous
import jax
import jax.numpy as jnp
from jax import lax
from jax.experimental import pallas as pl
from jax.experimental.pallas import tpu as pltpu
from jax.experimental.pallas import tpu_sc as plsc

N_IN = 500000
OUT = 1000000
C = 64
CP = 128
NW = 32
RPW = 31248
ZR = 124
NZ = RPW // ZR
TAIL = OUT - NW * RPW
IB = 50
GW = 8
WR = IB * GW
NG = N_IN // WR
SB = 48


def _sc_body(x_hbm, idx2_hbm, starts_hbm, out_hbm,
             zeros_v, idxw0, idxw1, xw0, xw1, starts_s, sem_z, sw0, sw1):
    c = lax.axis_index("c")
    s = lax.axis_index("s")
    wid = s * 2 + c
    base = wid * RPW

    pltpu.sync_copy(starts_hbm, starts_s)

    zvec = jnp.zeros((16,), jnp.float32)

    @pl.loop(0, ZR)
    def _(r):
        for j in range(C // 16):
            zeros_v[r, pl.ds(j * 16, 16)] = zvec

    zcopies = [
        pltpu.async_copy(
            zeros_v,
            out_hbm.at[pl.ds(base + k * ZR, ZR), pl.ds(0, C)],
            sem_z,
        )
        for k in range(NZ)
    ]
    for cp in zcopies:
        cp.wait()

    @pl.when(wid == NW - 1)
    def _():
        pltpu.async_copy(
            zeros_v.at[pl.ds(0, TAIL)],
            out_hbm.at[pl.ds(NW * RPW, TAIL), pl.ds(0, C)],
            sem_z,
        ).wait()

    sv = starts_s[pl.ds(wid, 16)]
    lo = sv[0]
    hi = sv[1]
    g0 = lo // WR
    g1 = (hi + WR - 1) // WR

    idxw = (idxw0, idxw1)
    xw = (xw0, xw1)
    sw = (sw0, sw1)

    def issue_loads(r, g):
        pltpu.async_copy(idx2_hbm.at[pl.ds(g * GW, GW)], idxw[r], sw[r])
        pltpu.async_copy(x_hbm.at[pl.ds(g * WR, WR), pl.ds(0, C)],
                         xw[r].at[pl.ds(0, WR), pl.ds(0, C)], sw[r])

    def wait_loads(r):
        pltpu.make_async_copy(idx2_hbm.at[pl.ds(0, GW)], idxw[r],
                              sw[r]).wait()
        pltpu.make_async_copy(x_hbm.at[pl.ds(0, WR), pl.ds(0, C)],
                              xw[r].at[pl.ds(0, WR), pl.ds(0, C)],
                              sw[r]).wait()

    def halfwin(g, r):
        @pl.when(g < g1)
        def _():
            wait_loads(r)
            for j in range(GW):
                pltpu.sync_copy(xw[r].at[pl.ds(j * IB, IB)],
                                out_hbm.at[idxw[r].at[j]])

            @pl.when(g + 2 < g1)
            def _():
                issue_loads(r, g + 2)

    @pl.when(g0 < g1)
    def _():
        issue_loads(0, g0)

    @pl.when(g0 + 1 < g1)
    def _():
        issue_loads(1, g0 + 1)

    def pair(p, carry):
        g = g0 + 2 * p
        halfwin(g, 0)
        halfwin(g + 1, 1)
        return carry

    lax.fori_loop(0, (g1 - g0 + 1) // 2, pair, 0)


def kernel(x, idx, out_size):
    del out_size
    idx = idx.astype(jnp.int32)
    bounds = jnp.concatenate([
        jnp.arange(0, NW * RPW, RPW, dtype=jnp.int32),
        jnp.array([OUT], dtype=jnp.int32),
    ])
    starts = jnp.searchsorted(idx, bounds).astype(jnp.int32)
    starts = jnp.zeros((SB,), jnp.int32).at[: NW + 1].set(starts)
    idx2 = idx.reshape(NG * GW, IB)

    mesh = plsc.VectorSubcoreMesh(core_axis_name="c", subcore_axis_name="s")
    xp = jnp.pad(x, ((0, 0), (0, CP - C)))
    run = pl.kernel(
        _sc_body,
        out_type=jax.ShapeDtypeStruct((OUT, CP), jnp.float32),
        mesh=mesh,
        compiler_params=pltpu.CompilerParams(use_tc_tiling_on_sc=False),
        scratch_types=[
            pltpu.VMEM((ZR, C), jnp.float32),
            pltpu.VMEM((GW, IB), jnp.int32),
            pltpu.VMEM((GW, IB), jnp.int32),
            pltpu.VMEM((WR, CP), jnp.float32),
            pltpu.VMEM((WR, CP), jnp.float32),
            pltpu.VMEM((SB,), jnp.int32),
            pltpu.SemaphoreType.DMA,
            pltpu.SemaphoreType.DMA,
            pltpu.SemaphoreType.DMA,
        ],
    )
    outp = run(xp, idx2, starts)
    return outp[:, :C]
